# baseline (speedup 1.0000x reference)
import functools

import jax
import jax.numpy as jnp
from jax import lax
from jax.experimental import pallas as pl
from jax.experimental.pallas import tpu as pltpu
from jax.experimental.pallas import tpu_sc as plsc

B = 16384
F = 26
FPAD = 128
VOCAB = 1000000

NC = 2
NS = 16
NW = NC * NS
CHUNK = B // NW
NIDX = CHUNK * F
L = 16
G = CHUNK // 8
QPG = 8 * F
VPG = QPG // L


def _body(tab_hbm, xp_hbm, out_hbm, xpad_v, idx_v, vals_v, out_v, sem):
    wid = lax.axis_index("s") * NC + lax.axis_index("c")
    base = wid * CHUNK

    pltpu.sync_copy(xp_hbm.at[pl.ds(base, CHUNK), :], xpad_v)

    iota = lax.iota(jnp.int32, L)

    def t_body(t, _):
        row0 = t * 8
        dst0 = t * QPG
        for v in range(VPG):
            q = iota + v * L
            acc_row = row0 + jnp.bitwise_and(q, 7)
            acc_col = jnp.right_shift(q, 3)
            vals = plsc.load_gather(xpad_v, [acc_row, acc_col])
            idx_v[pl.ds(dst0 + v * L, L)] = vals
        return _

    lax.fori_loop(0, G, t_body, None)

    pltpu.async_copy(tab_hbm.at[idx_v], vals_v, sem).wait()

    lanevec = jnp.right_shift(iota, 3) * QPG + jnp.bitwise_and(iota, 7)

    def g_body(g, _):
        pos0 = lanevec + g * (2 * QPG)
        acc = jnp.zeros((L,), jnp.float32)
        for c in range(F):
            acc = acc + plsc.load_gather(vals_v, [pos0 + c * 8])
        out_v[pl.ds(g * L, L)] = 1.0 / (1.0 + jnp.exp(-acc))
        return _

    lax.fori_loop(0, CHUNK // L, g_body, None)

    pltpu.sync_copy(out_v, out_hbm.at[pl.ds(base, CHUNK)])


_sc_call = functools.partial(
    pl.kernel,
    out_type=jax.ShapeDtypeStruct((B,), jnp.float32),
    mesh=plsc.VectorSubcoreMesh(
        core_axis_name="c", subcore_axis_name="s",
        num_cores=NC, num_subcores=NS,
    ),
    compiler_params=pltpu.CompilerParams(needs_layout_passes=False),
    scratch_types=[
        pltpu.VMEM((CHUNK, FPAD), jnp.int32),
        pltpu.VMEM((NIDX,), jnp.int32),
        pltpu.VMEM((NIDX,), jnp.float32),
        pltpu.VMEM((CHUNK,), jnp.float32),
        pltpu.SemaphoreType.DMA,
    ],
)(_body)


@jax.jit
def kernel(x, table):
    xp = jnp.pad(x, ((0, 0), (0, FPAD - F)))
    return _sc_call(table.reshape(VOCAB), xp)


# speedup vs baseline: 1.0001x; 1.0001x over previous
import functools

import jax
import jax.numpy as jnp
from jax import lax
from jax.experimental import pallas as pl
from jax.experimental.pallas import tpu as pltpu
from jax.experimental.pallas import tpu_sc as plsc

B = 16384
F = 26
FPAD = 128
VOCAB = 1000000

NC = 2
NS = 16
NW = NC * NS
CHUNK = B // NW
NIDX = CHUNK * F
L = 16
G = CHUNK // 8
QPG = 8 * F
VPG = QPG // L


def _body(tab_hbm, xp_hbm, out_hbm, xpad_v, idx_v, vals_v, out_v, sem):
    wid = lax.axis_index("s") * NC + lax.axis_index("c")
    base = wid * CHUNK

    pltpu.sync_copy(xp_hbm.at[pl.ds(base * FPAD, CHUNK * FPAD)], xpad_v)

    iota = lax.iota(jnp.int32, L)

    def t_body(t, _):
        row0 = t * 8
        dst0 = t * QPG
        for v in range(VPG):
            q = iota + v * L
            src = row0 * FPAD + jnp.bitwise_and(q, 7) * FPAD + jnp.right_shift(q, 3)
            idx_v[pl.ds(dst0 + v * L, L)] = plsc.load_gather(xpad_v, [src])
        return _

    lax.fori_loop(0, G, t_body, None)

    pltpu.async_copy(tab_hbm.at[idx_v], vals_v, sem).wait()

    lanevec = jnp.right_shift(iota, 3) * QPG + jnp.bitwise_and(iota, 7)

    def g_body(g, _):
        pos0 = lanevec + g * (2 * QPG)
        acc = jnp.zeros((L,), jnp.float32)
        for c in range(F):
            acc = acc + plsc.load_gather(vals_v, [pos0 + c * 8])
        out_v[pl.ds(g * L, L)] = 1.0 / (1.0 + jnp.exp(-acc))
        return _

    lax.fori_loop(0, CHUNK // L, g_body, None)

    pltpu.sync_copy(out_v, out_hbm.at[pl.ds(base, CHUNK)])


_sc_call = functools.partial(
    pl.kernel,
    out_type=jax.ShapeDtypeStruct((B,), jnp.float32),
    mesh=plsc.VectorSubcoreMesh(
        core_axis_name="c", subcore_axis_name="s",
        num_cores=NC, num_subcores=NS,
    ),
    compiler_params=pltpu.CompilerParams(needs_layout_passes=False),
    scratch_types=[
        pltpu.VMEM((CHUNK * FPAD,), jnp.int32),
        pltpu.VMEM((NIDX,), jnp.int32),
        pltpu.VMEM((NIDX,), jnp.float32),
        pltpu.VMEM((CHUNK,), jnp.float32),
        pltpu.SemaphoreType.DMA,
    ],
)(_body)


@jax.jit
def kernel(x, table):
    xp = jnp.pad(x, ((0, 0), (0, FPAD - F))).reshape(B * FPAD)
    return _sc_call(table.reshape(VOCAB), xp)


# speedup vs baseline: 1.1900x; 1.1899x over previous
import functools

import jax
import jax.numpy as jnp
from jax import lax
from jax.experimental import pallas as pl
from jax.experimental.pallas import tpu as pltpu
from jax.experimental.pallas import tpu_sc as plsc

B = 16384
F = 26
VOCAB = 1000000
VPAD = 1000064

NC = 2
NS = 16
NW = NC * NS
CHUNK = B // NW
NIDX = CHUNK * F
L = 16


def _body(tf_hbm, xr_hbm, out_hbm, idx_v, vals_v, out_v, sem):
    wid = lax.axis_index("s") * NC + lax.axis_index("c")

    pltpu.sync_copy(xr_hbm.at[wid], idx_v)

    pltpu.async_copy(tf_hbm.at[idx_v], vals_v, sem).wait()

    def g_body(g, _):
        base = g * L
        acc = jnp.zeros((L,), jnp.float32)
        for f in range(F):
            acc = acc + vals_v[pl.ds(f * CHUNK + base, L)]
        out_v[pl.ds(base, L)] = 1.0 / (1.0 + jnp.exp(-acc))
        return _

    lax.fori_loop(0, CHUNK // L, g_body, None)

    pltpu.sync_copy(out_v, out_hbm.at[pl.ds(wid * CHUNK, CHUNK)])


_sc_call = functools.partial(
    pl.kernel,
    out_type=jax.ShapeDtypeStruct((B,), jnp.float32),
    mesh=plsc.VectorSubcoreMesh(
        core_axis_name="c", subcore_axis_name="s",
        num_cores=NC, num_subcores=NS,
    ),
    compiler_params=pltpu.CompilerParams(needs_layout_passes=False),
    scratch_types=[
        pltpu.VMEM((NIDX,), jnp.int32),
        pltpu.VMEM((NIDX,), jnp.float32),
        pltpu.VMEM((CHUNK,), jnp.float32),
        pltpu.SemaphoreType.DMA,
    ],
)(_body)


@jax.jit
def kernel(x, table):
    xr = x.reshape(NW, CHUNK, F).transpose(0, 2, 1).reshape(NW, NIDX)
    tt = jnp.pad(table, ((0, VPAD - VOCAB), (0, 0)))
    tt = tt.reshape(VPAD // 128, 128).reshape(VPAD)
    return _sc_call(tt, xr)
